# Initial kernel scaffold; baseline (speedup 1.0000x reference)
import jax
import jax.numpy as jnp
from jax.experimental import pallas as pl


def kernel(pc, feat, params):
    raise NotImplementedError("write your pallas kernel here")


# speedup vs baseline: 4.8262x; 4.8262x over previous
import functools

import jax
import jax.numpy as jnp
from jax.experimental import pallas as pl
from jax.experimental.pallas import tpu as pltpu

_NPOINT = 1024
_RADIUS = [0.1, 0.2, 0.4]
_NSAMPLES = [16, 32, 64]
_EPS = 1e-5

_HI = jax.lax.Precision.HIGHEST
_S_TILE = 32
_N_CHUNK = 512
_T_TILE = 4096
_S_TILE_MAX = 128


def _fps_kernel(pc_ref, idx_ref, dists_ref):
    pc = pc_ref[...]
    B, _, N = pc.shape
    dists_ref[...] = jnp.full((B, N), 1e10, jnp.float32)
    iota = jax.lax.broadcasted_iota(jnp.int32, (B, N), 1)
    iota_s = jax.lax.broadcasted_iota(jnp.int32, idx_ref.shape, 1)
    idx_ref[...] = jnp.zeros(idx_ref.shape, jnp.int32)

    def body(i, farthest):
        idx_ref[...] = jnp.where(iota_s == i, farthest, idx_ref[...])
        onehot = (iota == farthest).astype(jnp.float32)
        dtot = None
        for c in range(3):
            xc = pc[:, c, :]
            cc = jnp.sum(xc * onehot, axis=-1, keepdims=True)
            diff = xc - cc
            sq = diff * diff
            dtot = sq if dtot is None else dtot + sq
        dnew = jnp.minimum(dists_ref[...], dtot)
        dists_ref[...] = dnew
        return jnp.argmax(dnew, axis=-1).astype(jnp.int32)[:, None]

    jax.lax.fori_loop(0, _NPOINT, body, jnp.zeros((B, 1), jnp.int32))


def _group_l1_kernel(idx_ref, pc_ref, vals_ref, w1_ref,
                     y1_ref, ctr_ref, part_ref, *, ns, r2):
    S_t = _S_TILE
    C = vals_ref.shape[1]
    N = pc_ref.shape[2]
    pcm = pc_ref[0]
    idx = idx_ref[0]

    iota_n = jax.lax.broadcasted_iota(jnp.int32, (S_t, N), 1)
    onehot = (iota_n == idx).astype(jnp.float32)
    centers = jax.lax.dot_general(
        onehot, pcm, (((1,), (1,)), ((), ())), precision=_HI)
    ctr_ref[0] = centers

    d2 = None
    for c in range(3):
        diff = centers[:, c:c + 1] - pcm[c, :][None, :]
        sq = diff * diff
        d2 = sq if d2 is None else d2 + sq
    mask = (d2 <= r2).astype(jnp.float32)

    n_chunks = N // _N_CHUNK
    tri = (jax.lax.broadcasted_iota(jnp.int32, (_N_CHUNK, _N_CHUNK), 0)
           < jax.lax.broadcasted_iota(jnp.int32, (_N_CHUNK, _N_CHUNK), 1)
           ).astype(jnp.float32)
    kidx = jax.lax.broadcasted_iota(
        jnp.int32, (S_t, ns, _N_CHUNK), 1).astype(jnp.float32)

    acc = jnp.zeros((S_t * ns, C), jnp.float32)
    base = jnp.zeros((S_t, 1), jnp.float32)
    for ci in range(n_chunks):
        lo = ci * _N_CHUNK
        mask_c = mask[:, lo:lo + _N_CHUNK]
        rank = base + jax.lax.dot_general(
            mask_c, tri, (((1,), (0,)), ((), ())), precision=_HI)
        q3 = mask_c[:, None, :] * (rank[:, None, :] == kidx).astype(jnp.float32)
        q = q3.reshape(S_t * ns, _N_CHUNK)
        vals_c = vals_ref[0, :, lo:lo + _N_CHUNK]
        acc = acc + jax.lax.dot_general(
            q, vals_c, (((1,), (1,)), ((), ())), precision=_HI)
        base = base + jnp.sum(mask_c, axis=1, keepdims=True)

    g3 = acc.reshape(S_t, ns, C)
    ctr_full = jnp.concatenate(
        [centers, jnp.zeros((S_t, C - 3), jnp.float32)], axis=1)
    g3 = g3 - ctr_full[:, None, :]

    y1 = jax.lax.dot_general(
        g3.reshape(S_t * ns, C), w1_ref[...],
        (((1,), (1,)), ((), ())), precision=_HI)

    rows = S_t * ns
    iota_r0 = jax.lax.broadcasted_iota(jnp.int32, (rows, 1), 0)
    sel_first = (jax.lax.broadcasted_iota(jnp.int32, (S_t, rows), 1)
                 == jax.lax.broadcasted_iota(jnp.int32, (S_t, rows), 0) * ns
                 ).astype(jnp.float32)
    rep = (jax.lax.broadcasted_iota(jnp.int32, (rows, S_t), 1)
           == jax.lax.broadcasted_iota(jnp.int32, (rows, S_t), 0) // ns
           ).astype(jnp.float32)
    countrow = jax.lax.dot_general(
        rep, base, (((1,), (0,)), ((), ())), precision=_HI)
    kmod = (iota_r0 % ns).astype(jnp.float32)
    first = jax.lax.dot_general(
        sel_first, y1, (((1,), (0,)), ((), ())), precision=_HI)
    firstfull = jax.lax.dot_general(
        rep, first, (((1,), (0,)), ((), ())), precision=_HI)
    y1 = jnp.where(kmod < countrow, y1, firstfull)
    y1_ref[0] = y1

    @pl.when((pl.program_id(0) == 0) & (pl.program_id(1) == 0))
    def _init():
        part_ref[...] = jnp.zeros_like(part_ref)

    s1 = jnp.sum(y1, axis=0, keepdims=True)
    s2 = jnp.sum(y1 * y1, axis=0, keepdims=True)
    part_ref[...] += jnp.concatenate([s1, s2], axis=0)


def _bn_mm_kernel(y_ref, a_ref, c_ref, w_ref, o_ref, part_ref):
    x = jnp.maximum(y_ref[0] * a_ref[...] + c_ref[...], 0.0)
    y = jax.lax.dot_general(
        x, w_ref[...], (((1,), (1,)), ((), ())), precision=_HI)
    o_ref[0] = y

    @pl.when((pl.program_id(0) == 0) & (pl.program_id(1) == 0))
    def _init():
        part_ref[...] = jnp.zeros_like(part_ref)

    s1 = jnp.sum(y, axis=0, keepdims=True)
    s2 = jnp.sum(y * y, axis=0, keepdims=True)
    part_ref[...] += jnp.concatenate([s1, s2], axis=0)


def _bn_max_kernel(y_ref, a_ref, c_ref, o_ref, *, ns):
    x = jnp.maximum(y_ref[0] * a_ref[...] + c_ref[...], 0.0)
    rows, O = x.shape
    x3 = x.reshape(rows // ns, ns, O)
    g = ns
    while g > 1:
        h = g // 2
        x3 = jnp.maximum(x3[:, :h, :], x3[:, h:g, :])
        g = h
    o_ref[0] = x3[:, 0, :]


def _stats_to_affine(part, count, gamma, beta):
    mean = part[0] / count
    var = jnp.maximum(part[1] / count - mean * mean, 0.0)
    a = gamma / jnp.sqrt(var + _EPS)
    c = beta - mean * a
    return a[None, :], c[None, :]


def kernel(pc, feat, params):
    B, _, N = pc.shape
    S = _NPOINT

    fps_idx = pl.pallas_call(
        _fps_kernel,
        out_shape=jax.ShapeDtypeStruct((B, S), jnp.int32),
        scratch_shapes=[pltpu.VMEM((B, N), jnp.float32)],
    )(pc)

    vals = jnp.concatenate([pc, feat], axis=1)
    C = vals.shape[1]
    st_grid = S // _S_TILE

    pc_sample = None
    cat_feat = []
    for r, ns, layer_params in zip(_RADIUS, _NSAMPLES, params):
        w1, g1, b1 = layer_params[0]
        O1 = w1.shape[0]
        sn = S * ns

        y, ctr, part = pl.pallas_call(
            functools.partial(_group_l1_kernel, ns=ns, r2=r * r),
            grid=(B, st_grid),
            in_specs=[
                pl.BlockSpec((1, _S_TILE, 1), lambda b, s: (b, s, 0)),
                pl.BlockSpec((1, 3, N), lambda b, s: (b, 0, 0)),
                pl.BlockSpec((1, C, N), lambda b, s: (b, 0, 0)),
                pl.BlockSpec((O1, C), lambda b, s: (0, 0)),
            ],
            out_specs=[
                pl.BlockSpec((1, _S_TILE * ns, O1), lambda b, s: (b, s, 0)),
                pl.BlockSpec((1, _S_TILE, 3), lambda b, s: (b, s, 0)),
                pl.BlockSpec((2, O1), lambda b, s: (0, 0)),
            ],
            out_shape=[
                jax.ShapeDtypeStruct((B, sn, O1), jnp.float32),
                jax.ShapeDtypeStruct((B, S, 3), jnp.float32),
                jax.ShapeDtypeStruct((2, O1), jnp.float32),
            ],
        )(fps_idx[:, :, None], pc, vals, w1)
        if pc_sample is None:
            pc_sample = jnp.transpose(ctr, (0, 2, 1))

        count = float(B * sn)
        gamma, beta = g1, b1
        for (w, g_next, b_next) in layer_params[1:]:
            a, cc = _stats_to_affine(part, count, gamma, beta)
            Oi, On = w.shape[1], w.shape[0]
            y, part = pl.pallas_call(
                _bn_mm_kernel,
                grid=(B, sn // _T_TILE),
                in_specs=[
                    pl.BlockSpec((1, _T_TILE, Oi), lambda b, t: (b, t, 0)),
                    pl.BlockSpec((1, Oi), lambda b, t: (0, 0)),
                    pl.BlockSpec((1, Oi), lambda b, t: (0, 0)),
                    pl.BlockSpec((On, Oi), lambda b, t: (0, 0)),
                ],
                out_specs=[
                    pl.BlockSpec((1, _T_TILE, On), lambda b, t: (b, t, 0)),
                    pl.BlockSpec((2, On), lambda b, t: (0, 0)),
                ],
                out_shape=[
                    jax.ShapeDtypeStruct((B, sn, On), jnp.float32),
                    jax.ShapeDtypeStruct((2, On), jnp.float32),
                ],
            )(y, a, cc, w)
            gamma, beta = g_next, b_next

        a, cc = _stats_to_affine(part, count, gamma, beta)
        Of = y.shape[-1]
        out = pl.pallas_call(
            functools.partial(_bn_max_kernel, ns=ns),
            grid=(B, S // _S_TILE_MAX),
            in_specs=[
                pl.BlockSpec((1, _S_TILE_MAX * ns, Of), lambda b, t: (b, t, 0)),
                pl.BlockSpec((1, Of), lambda b, t: (0, 0)),
                pl.BlockSpec((1, Of), lambda b, t: (0, 0)),
            ],
            out_specs=pl.BlockSpec((1, _S_TILE_MAX, Of), lambda b, t: (b, t, 0)),
            out_shape=jax.ShapeDtypeStruct((B, S, Of), jnp.float32),
        )(y, a, cc)
        cat_feat.append(jnp.transpose(out, (0, 2, 1)))

    return (pc_sample, jnp.concatenate(cat_feat, axis=1))
